# Initial kernel scaffold; baseline (speedup 1.0000x reference)
import jax
import jax.numpy as jnp
from jax.experimental import pallas as pl


def kernel(x, edge_index, graph_ids, W_emb, W_gcn0, W_gcn1, W_gcn2, W_p1, W_p2, b_p2):
    raise NotImplementedError("write your pallas kernel here")


# speedup vs baseline: 5.7003x; 5.7003x over previous
import functools

import jax
import jax.numpy as jnp
from jax import lax
from jax.experimental import pallas as pl
from jax.experimental.pallas import tpu as pltpu
from jax.experimental.pallas import tpu_sc as plsc

N = 50000
E = 800000
B = 128
NC = 2
NS = 16

K = 100
IB = 1000
NI = IB // K

WB = 3128
WB_LAST = N - 15 * WB

R = 5000
GRID = N // R


def _zero_acc(zeros_hbm, acc, s):
    @pl.when(s < NS - 1)
    def _():
        pltpu.sync_copy(zeros_hbm, acc.at[pl.ds(s * WB, WB)])

    @pl.when(s == NS - 1)
    def _():
        pltpu.sync_copy(zeros_hbm.at[pl.ds(0, WB_LAST)],
                        acc.at[pl.ds((NS - 1) * WB, WB_LAST)])


def _writeback(acc, out_hbm, s):
    @pl.when(s < NS - 1)
    def _():
        pltpu.sync_copy(acc.at[pl.ds(s * WB, WB)],
                        out_hbm.at[pl.ds(s * WB, WB)])

    @pl.when(s == NS - 1)
    def _():
        pltpu.sync_copy(acc.at[pl.ds((NS - 1) * WB, WB_LAST)],
                        out_hbm.at[pl.ds((NS - 1) * WB, WB_LAST)])


def _edge_loop(h_hbm, src2_hbm, dst2_hbm, acc, sidx, didx, rows, sem,
               base_blk, n_outer):
    def outer(i, carry):
        blk = base_blk + i
        pltpu.sync_copy(src2_hbm.at[blk], sidx)
        pltpu.sync_copy(dst2_hbm.at[blk], didx)
        for j in range(NI):
            pltpu.async_copy(h_hbm.at[sidx.at[j]], rows, sem).wait()
            pltpu.sync_copy(rows, acc.at[didx.at[j]], add=True)
        return carry

    lax.fori_loop(0, n_outer, outer, 0)


def _segsum8_body(h_hbm, src2_hbm, dst2_hbm, zeros_hbm,
                  out0_hbm, out1_hbm, acc, sidx, didx, rows, sem):
    c = lax.axis_index("c")
    s = lax.axis_index("s")
    _zero_acc(zeros_hbm, acc, s)
    plsc.subcore_barrier()
    t_edges = E // (NC * NS)
    base_blk = (c * NS + s) * (t_edges // IB)
    _edge_loop(h_hbm, src2_hbm, dst2_hbm, acc, sidx, didx, rows, sem,
               base_blk, t_edges // IB)
    plsc.subcore_barrier()

    @pl.when(c == 0)
    def _():
        _writeback(acc, out0_hbm, s)

    @pl.when(c == 1)
    def _():
        _writeback(acc, out1_hbm, s)


def _segsum32_body(m0_hbm, m1_hbm, src2_hbm, dst2_hbm, zeros_hbm,
                   out0_hbm, out1_hbm, acc, sidx, didx, rows, sem):
    c = lax.axis_index("c")
    s = lax.axis_index("s")
    _zero_acc(zeros_hbm, acc, s)
    plsc.subcore_barrier()
    t_edges = E // NS
    base_blk = s * (t_edges // IB)
    n_outer = t_edges // IB

    @pl.when(c == 0)
    def _():
        _edge_loop(m0_hbm, src2_hbm, dst2_hbm, acc, sidx, didx, rows, sem,
                   base_blk, n_outer)

    @pl.when(c == 1)
    def _():
        _edge_loop(m1_hbm, src2_hbm, dst2_hbm, acc, sidx, didx, rows, sem,
                   base_blk, n_outer)

    plsc.subcore_barrier()

    @pl.when(c == 0)
    def _():
        _writeback(acc, out0_hbm, s)

    @pl.when(c == 1)
    def _():
        _writeback(acc, out1_hbm, s)


def _sc_mesh():
    return plsc.VectorSubcoreMesh(core_axis_name="c", subcore_axis_name="s",
                                  num_cores=NC, num_subcores=NS)


def _segsum8(h0, src2, dst2, zeros8):
    f = pl.kernel(
        _segsum8_body,
        out_type=(jax.ShapeDtypeStruct((N, 8), jnp.float32),
                  jax.ShapeDtypeStruct((N, 8), jnp.float32)),
        mesh=_sc_mesh(),
        compiler_params=pltpu.CompilerParams(use_tc_tiling_on_sc=False),
        scratch_types=[
            pltpu.VMEM_SHARED((N, 8), jnp.float32),
            pltpu.VMEM((NI, K), jnp.int32),
            pltpu.VMEM((NI, K), jnp.int32),
            pltpu.VMEM((K, 8), jnp.float32),
            pltpu.SemaphoreType.DMA,
        ],
    )
    return f(h0, src2, dst2, zeros8)


def _segsum32(m0, m1, src2, dst2, zeros32):
    f = pl.kernel(
        _segsum32_body,
        out_type=(jax.ShapeDtypeStruct((N, 32), jnp.float32),
                  jax.ShapeDtypeStruct((N, 32), jnp.float32)),
        mesh=_sc_mesh(),
        compiler_params=pltpu.CompilerParams(use_tc_tiling_on_sc=False),
        scratch_types=[
            pltpu.VMEM_SHARED((N, 32), jnp.float32),
            pltpu.VMEM((NI, K), jnp.int32),
            pltpu.VMEM((NI, K), jnp.int32),
            pltpu.VMEM((K, 32), jnp.float32),
            pltpu.SemaphoreType.DMA,
        ],
    )
    return f(m0, m1, src2, dst2, zeros32)


def _emb_body(x_ref, w_ref, o_ref):
    o_ref[...] = jnp.dot(x_ref[...], w_ref[...],
                         preferred_element_type=jnp.float32)


def _tc_emb(x, w_emb):
    return pl.pallas_call(
        _emb_body,
        grid=(GRID,),
        in_specs=[pl.BlockSpec((R, 27), lambda i: (i, 0)),
                  pl.BlockSpec((27, 8), lambda i: (0, 0))],
        out_specs=pl.BlockSpec((R, 8), lambda i: (i, 0)),
        out_shape=jax.ShapeDtypeStruct((N, 8), jnp.float32),
    )(x, w_emb)


def _layer0_body(a0_ref, a1_ref, w_ref, o0_ref, o1_ref):
    h = jnp.maximum(jnp.dot(a0_ref[...] + a1_ref[...], w_ref[...],
                            preferred_element_type=jnp.float32), 0.0)
    o0_ref[...] = h[:, :32]
    o1_ref[...] = h[:, 32:]


def _tc_layer0(a0, a1, w):
    return pl.pallas_call(
        _layer0_body,
        grid=(GRID,),
        in_specs=[pl.BlockSpec((R, 8), lambda i: (i, 0)),
                  pl.BlockSpec((R, 8), lambda i: (i, 0)),
                  pl.BlockSpec((8, 64), lambda i: (0, 0))],
        out_specs=(pl.BlockSpec((R, 32), lambda i: (i, 0)),
                   pl.BlockSpec((R, 32), lambda i: (i, 0))),
        out_shape=(jax.ShapeDtypeStruct((N, 32), jnp.float32),
                   jax.ShapeDtypeStruct((N, 32), jnp.float32)),
    )(a0, a1, w)


def _layer_body(g0_ref, g1_ref, h0_ref, h1_ref, w_ref, o0_ref, o1_ref):
    g = jnp.concatenate([g0_ref[...], g1_ref[...]], axis=1)
    hp = jnp.concatenate([h0_ref[...], h1_ref[...]], axis=1)
    h = jnp.maximum(jnp.dot(g, w_ref[...],
                            preferred_element_type=jnp.float32) + hp, 0.0)
    o0_ref[...] = h[:, :32]
    o1_ref[...] = h[:, 32:]


def _tc_layer(g0, g1, h0, h1, w):
    return pl.pallas_call(
        _layer_body,
        grid=(GRID,),
        in_specs=[pl.BlockSpec((R, 32), lambda i: (i, 0))] * 4
        + [pl.BlockSpec((64, 64), lambda i: (0, 0))],
        out_specs=(pl.BlockSpec((R, 32), lambda i: (i, 0)),
                   pl.BlockSpec((R, 32), lambda i: (i, 0))),
        out_shape=(jax.ShapeDtypeStruct((N, 32), jnp.float32),
                   jax.ShapeDtypeStruct((N, 32), jnp.float32)),
    )(g0, g1, h0, h1, w)


def _final_body(g0_ref, g1_ref, h0_ref, h1_ref, w_ref, ids_ref,
                wp1_ref, wp2_ref, b_ref, o_ref, pooled):
    i = pl.program_id(0)
    g = jnp.concatenate([g0_ref[...], g1_ref[...]], axis=1)
    hp = jnp.concatenate([h0_ref[...], h1_ref[...]], axis=1)
    h = jnp.maximum(jnp.dot(g, w_ref[...],
                            preferred_element_type=jnp.float32) + hp, 0.0)
    ids = ids_ref[0]
    onehot = (lax.broadcasted_iota(jnp.int32, (B, R), 0) == ids
              ).astype(jnp.float32)

    @pl.when(i == 0)
    def _():
        pooled[...] = jnp.zeros_like(pooled)

    pooled[...] += jnp.dot(onehot, h, preferred_element_type=jnp.float32)

    @pl.when(i == GRID - 1)
    def _():
        z = jnp.maximum(jnp.dot(pooled[...], wp1_ref[...],
                                preferred_element_type=jnp.float32), 0.0)
        o_ref[...] = (jnp.dot(z, wp2_ref[...],
                              preferred_element_type=jnp.float32)
                      + b_ref[...])


def _tc_final(g0, g1, h0, h1, w, ids3, wp1, wp2, b2):
    return pl.pallas_call(
        _final_body,
        grid=(GRID,),
        in_specs=[pl.BlockSpec((R, 32), lambda i: (i, 0))] * 4
        + [pl.BlockSpec((64, 64), lambda i: (0, 0)),
           pl.BlockSpec((1, 1, R), lambda i: (i, 0, 0)),
           pl.BlockSpec((64, 32), lambda i: (0, 0)),
           pl.BlockSpec((32, 1), lambda i: (0, 0)),
           pl.BlockSpec((1, 1), lambda i: (0, 0))],
        out_specs=pl.BlockSpec((B, 1), lambda i: (0, 0)),
        out_shape=jax.ShapeDtypeStruct((B, 1), jnp.float32),
        scratch_shapes=[pltpu.VMEM((B, 64), jnp.float32)],
    )(g0, g1, h0, h1, w, ids3, wp1, wp2, b2)


def kernel(x, edge_index, graph_ids, W_emb, W_gcn0, W_gcn1, W_gcn2,
           W_p1, W_p2, b_p2):
    src2 = edge_index[0].reshape(E // IB, NI, K)
    dst2 = edge_index[1].reshape(E // IB, NI, K)
    zeros8 = jnp.zeros((WB, 8), jnp.float32)
    zeros32 = jnp.zeros((WB, 32), jnp.float32)
    ids3 = graph_ids.reshape(GRID, 1, R)
    b2 = b_p2.reshape(1, 1)

    h0 = _tc_emb(x, W_emb)
    a0, a1 = _segsum8(h0, src2, dst2, zeros8)
    h1_0, h1_1 = _tc_layer0(a0, a1, W_gcn0)
    g1_0, g1_1 = _segsum32(h1_0, h1_1, src2, dst2, zeros32)
    h2_0, h2_1 = _tc_layer(g1_0, g1_1, h1_0, h1_1, W_gcn1)
    g2_0, g2_1 = _segsum32(h2_0, h2_1, src2, dst2, zeros32)
    return _tc_final(g2_0, g2_1, h2_0, h2_1, W_gcn2, ids3, W_p1, W_p2, b2)


# speedup vs baseline: 10.0781x; 1.7680x over previous
import functools

import jax
import jax.numpy as jnp
from jax import lax
from jax.experimental import pallas as pl
from jax.experimental.pallas import tpu as pltpu
from jax.experimental.pallas import tpu_sc as plsc

N = 50000
E = 800000
B = 128
NC = 2
NS = 16

K = 125
NI = 8
P = 4

WB = 3128
WB_LAST = N - 15 * WB

R = 5000
GRID = N // R


def _zero_acc(zeros_hbm, acc, s):
    @pl.when(s < NS - 1)
    def _():
        pltpu.sync_copy(zeros_hbm, acc.at[pl.ds(s * WB, WB)])

    @pl.when(s == NS - 1)
    def _():
        pltpu.sync_copy(zeros_hbm.at[pl.ds(0, WB_LAST)],
                        acc.at[pl.ds((NS - 1) * WB, WB_LAST)])


def _writeback(acc, out_hbm, s):
    @pl.when(s < NS - 1)
    def _():
        pltpu.sync_copy(acc.at[pl.ds(s * WB, WB)],
                        out_hbm.at[pl.ds(s * WB, WB)])

    @pl.when(s == NS - 1)
    def _():
        pltpu.sync_copy(acc.at[pl.ds((NS - 1) * WB, WB_LAST)],
                        out_hbm.at[pl.ds((NS - 1) * WB, WB_LAST)])


def _edge_loop(h_hbm, src2_hbm, dst2_hbm, acc, sidx, didx, rows, sems,
               base_blk, n_outer):
    def outer(i, carry):
        blk = base_blk + i
        pltpu.sync_copy(src2_hbm.at[blk], sidx)
        pltpu.sync_copy(dst2_hbm.at[blk], didx)
        cps = [None] * NI
        for j in range(P):
            cps[j] = pltpu.async_copy(h_hbm.at[sidx.at[j]], rows.at[j % P],
                                      sems.at[j % P])
        for j in range(NI):
            cps[j].wait()
            pltpu.sync_copy(rows.at[j % P], acc.at[didx.at[j]], add=True)
            if j + P < NI:
                cps[j + P] = pltpu.async_copy(h_hbm.at[sidx.at[j + P]],
                                              rows.at[j % P], sems.at[j % P])
        return carry

    lax.fori_loop(0, n_outer, outer, 0)


def _segsum8_body(h_hbm, src2_hbm, dst2_hbm, zeros_hbm,
                  out0_hbm, out1_hbm, acc, sidx, didx, rows, sems):
    c = lax.axis_index("c")
    s = lax.axis_index("s")
    _zero_acc(zeros_hbm, acc, s)
    plsc.subcore_barrier()
    t_edges = E // (NC * NS)
    n_outer = t_edges // (NI * K)
    base_blk = (c * NS + s) * n_outer
    _edge_loop(h_hbm, src2_hbm, dst2_hbm, acc, sidx, didx, rows, sems,
               base_blk, n_outer)
    plsc.subcore_barrier()

    @pl.when(c == 0)
    def _():
        _writeback(acc, out0_hbm, s)

    @pl.when(c == 1)
    def _():
        _writeback(acc, out1_hbm, s)


def _segsum32_body(m0_hbm, m1_hbm, src2_hbm, dst2_hbm, zeros_hbm,
                   out0_hbm, out1_hbm, acc, sidx, didx, rows, sems):
    c = lax.axis_index("c")
    s = lax.axis_index("s")
    _zero_acc(zeros_hbm, acc, s)
    plsc.subcore_barrier()
    t_edges = E // NS
    n_outer = t_edges // (NI * K)
    base_blk = s * n_outer

    @pl.when(c == 0)
    def _():
        _edge_loop(m0_hbm, src2_hbm, dst2_hbm, acc, sidx, didx, rows, sems,
                   base_blk, n_outer)

    @pl.when(c == 1)
    def _():
        _edge_loop(m1_hbm, src2_hbm, dst2_hbm, acc, sidx, didx, rows, sems,
                   base_blk, n_outer)

    plsc.subcore_barrier()

    @pl.when(c == 0)
    def _():
        _writeback(acc, out0_hbm, s)

    @pl.when(c == 1)
    def _():
        _writeback(acc, out1_hbm, s)


def _sc_mesh():
    return plsc.VectorSubcoreMesh(core_axis_name="c", subcore_axis_name="s",
                                  num_cores=NC, num_subcores=NS)


def _segsum8(h0, src2, dst2, zeros8):
    f = pl.kernel(
        _segsum8_body,
        out_type=(jax.ShapeDtypeStruct((N, 8), jnp.float32),
                  jax.ShapeDtypeStruct((N, 8), jnp.float32)),
        mesh=_sc_mesh(),
        compiler_params=pltpu.CompilerParams(use_tc_tiling_on_sc=False),
        scratch_types=[
            pltpu.VMEM_SHARED((N, 8), jnp.float32),
            pltpu.VMEM((NI, K), jnp.int32),
            pltpu.VMEM((NI, K), jnp.int32),
            pltpu.VMEM((P, K, 8), jnp.float32),
            pltpu.SemaphoreType.DMA((P,)),
        ],
    )
    return f(h0, src2, dst2, zeros8)


def _segsum32(m0, m1, src2, dst2, zeros32):
    f = pl.kernel(
        _segsum32_body,
        out_type=(jax.ShapeDtypeStruct((N, 32), jnp.float32),
                  jax.ShapeDtypeStruct((N, 32), jnp.float32)),
        mesh=_sc_mesh(),
        compiler_params=pltpu.CompilerParams(use_tc_tiling_on_sc=False),
        scratch_types=[
            pltpu.VMEM_SHARED((N, 32), jnp.float32),
            pltpu.VMEM((NI, K), jnp.int32),
            pltpu.VMEM((NI, K), jnp.int32),
            pltpu.VMEM((P, K, 32), jnp.float32),
            pltpu.SemaphoreType.DMA((P,)),
        ],
    )
    return f(m0, m1, src2, dst2, zeros32)


def _emb_body(x_ref, w_ref, o_ref):
    o_ref[...] = jnp.dot(x_ref[...], w_ref[...],
                         preferred_element_type=jnp.float32)


def _tc_emb(x, w_emb):
    return pl.pallas_call(
        _emb_body,
        grid=(GRID,),
        in_specs=[pl.BlockSpec((R, 27), lambda i: (i, 0)),
                  pl.BlockSpec((27, 8), lambda i: (0, 0))],
        out_specs=pl.BlockSpec((R, 8), lambda i: (i, 0)),
        out_shape=jax.ShapeDtypeStruct((N, 8), jnp.float32),
    )(x, w_emb)


def _layer0_body(a0_ref, a1_ref, w_ref, o0_ref, o1_ref):
    h = jnp.maximum(jnp.dot(a0_ref[...] + a1_ref[...], w_ref[...],
                            preferred_element_type=jnp.float32), 0.0)
    o0_ref[...] = h[:, :32]
    o1_ref[...] = h[:, 32:]


def _tc_layer0(a0, a1, w):
    return pl.pallas_call(
        _layer0_body,
        grid=(GRID,),
        in_specs=[pl.BlockSpec((R, 8), lambda i: (i, 0)),
                  pl.BlockSpec((R, 8), lambda i: (i, 0)),
                  pl.BlockSpec((8, 64), lambda i: (0, 0))],
        out_specs=(pl.BlockSpec((R, 32), lambda i: (i, 0)),
                   pl.BlockSpec((R, 32), lambda i: (i, 0))),
        out_shape=(jax.ShapeDtypeStruct((N, 32), jnp.float32),
                   jax.ShapeDtypeStruct((N, 32), jnp.float32)),
    )(a0, a1, w)


def _layer_body(g0_ref, g1_ref, h0_ref, h1_ref, w_ref, o0_ref, o1_ref):
    g = jnp.concatenate([g0_ref[...], g1_ref[...]], axis=1)
    hp = jnp.concatenate([h0_ref[...], h1_ref[...]], axis=1)
    h = jnp.maximum(jnp.dot(g, w_ref[...],
                            preferred_element_type=jnp.float32) + hp, 0.0)
    o0_ref[...] = h[:, :32]
    o1_ref[...] = h[:, 32:]


def _tc_layer(g0, g1, h0, h1, w):
    return pl.pallas_call(
        _layer_body,
        grid=(GRID,),
        in_specs=[pl.BlockSpec((R, 32), lambda i: (i, 0))] * 4
        + [pl.BlockSpec((64, 64), lambda i: (0, 0))],
        out_specs=(pl.BlockSpec((R, 32), lambda i: (i, 0)),
                   pl.BlockSpec((R, 32), lambda i: (i, 0))),
        out_shape=(jax.ShapeDtypeStruct((N, 32), jnp.float32),
                   jax.ShapeDtypeStruct((N, 32), jnp.float32)),
    )(g0, g1, h0, h1, w)


def _final_body(g0_ref, g1_ref, h0_ref, h1_ref, w_ref, ids_ref,
                wp1_ref, wp2_ref, b_ref, o_ref, pooled):
    i = pl.program_id(0)
    g = jnp.concatenate([g0_ref[...], g1_ref[...]], axis=1)
    hp = jnp.concatenate([h0_ref[...], h1_ref[...]], axis=1)
    h = jnp.maximum(jnp.dot(g, w_ref[...],
                            preferred_element_type=jnp.float32) + hp, 0.0)
    ids = ids_ref[0]
    onehot = (lax.broadcasted_iota(jnp.int32, (B, R), 0) == ids
              ).astype(jnp.float32)

    @pl.when(i == 0)
    def _():
        pooled[...] = jnp.zeros_like(pooled)

    pooled[...] += jnp.dot(onehot, h, preferred_element_type=jnp.float32)

    @pl.when(i == GRID - 1)
    def _():
        z = jnp.maximum(jnp.dot(pooled[...], wp1_ref[...],
                                preferred_element_type=jnp.float32), 0.0)
        o_ref[...] = (jnp.dot(z, wp2_ref[...],
                              preferred_element_type=jnp.float32)
                      + b_ref[...])


def _tc_final(g0, g1, h0, h1, w, ids3, wp1, wp2, b2):
    return pl.pallas_call(
        _final_body,
        grid=(GRID,),
        in_specs=[pl.BlockSpec((R, 32), lambda i: (i, 0))] * 4
        + [pl.BlockSpec((64, 64), lambda i: (0, 0)),
           pl.BlockSpec((1, 1, R), lambda i: (i, 0, 0)),
           pl.BlockSpec((64, 32), lambda i: (0, 0)),
           pl.BlockSpec((32, 1), lambda i: (0, 0)),
           pl.BlockSpec((1, 1), lambda i: (0, 0))],
        out_specs=pl.BlockSpec((B, 1), lambda i: (0, 0)),
        out_shape=jax.ShapeDtypeStruct((B, 1), jnp.float32),
        scratch_shapes=[pltpu.VMEM((B, 64), jnp.float32)],
    )(g0, g1, h0, h1, w, ids3, wp1, wp2, b2)


def kernel(x, edge_index, graph_ids, W_emb, W_gcn0, W_gcn1, W_gcn2,
           W_p1, W_p2, b_p2):
    src3 = edge_index[0].reshape(E // (NI * K), NI, K)
    dst3 = edge_index[1].reshape(E // (NI * K), NI, K)
    zeros8 = jnp.zeros((WB, 8), jnp.float32)
    zeros32 = jnp.zeros((WB, 32), jnp.float32)
    ids3 = graph_ids.reshape(GRID, 1, R)
    b2 = b_p2.reshape(1, 1)

    h0 = _tc_emb(x, W_emb)
    a0, a1 = _segsum8(h0, src3, dst3, zeros8)
    h1_0, h1_1 = _tc_layer0(a0, a1, W_gcn0)
    g1_0, g1_1 = _segsum32(h1_0, h1_1, src3, dst3, zeros32)
    h2_0, h2_1 = _tc_layer(g1_0, g1_1, h1_0, h1_1, W_gcn1)
    g2_0, g2_1 = _segsum32(h2_0, h2_1, src3, dst3, zeros32)
    return _tc_final(g2_0, g2_1, h2_0, h2_1, W_gcn2, ids3, W_p1, W_p2, b2)


# speedup vs baseline: 11.4957x; 1.1407x over previous
import functools

import jax
import jax.numpy as jnp
from jax import lax
from jax.experimental import pallas as pl
from jax.experimental.pallas import tpu as pltpu
from jax.experimental.pallas import tpu_sc as plsc

N = 50000
E = 800000
B = 128
NC = 2
NS = 16

K = 125
NI = 16
NI8 = 8
P = 4

WB = 3128
WB_LAST = N - 15 * WB

R = 5000
GRID = N // R


def _zero_acc(zeros_hbm, acc, s):
    @pl.when(s < NS - 1)
    def _():
        pltpu.sync_copy(zeros_hbm, acc.at[pl.ds(s * WB, WB)])

    @pl.when(s == NS - 1)
    def _():
        pltpu.sync_copy(zeros_hbm.at[pl.ds(0, WB_LAST)],
                        acc.at[pl.ds((NS - 1) * WB, WB_LAST)])


def _writeback(acc, out_hbm, s):
    @pl.when(s < NS - 1)
    def _():
        pltpu.sync_copy(acc.at[pl.ds(s * WB, WB)],
                        out_hbm.at[pl.ds(s * WB, WB)])

    @pl.when(s == NS - 1)
    def _():
        pltpu.sync_copy(acc.at[pl.ds((NS - 1) * WB, WB_LAST)],
                        out_hbm.at[pl.ds((NS - 1) * WB, WB_LAST)])


def _edge_loop(h_hbm, e3_hbm, acc, sidx, didx, rows, sems,
               base_blk, n_outer, ni):
    def outer(i, carry):
        r0 = (base_blk + i) * ni
        pltpu.sync_copy(e3_hbm.at[0].at[pl.ds(r0, ni)], sidx)
        pltpu.sync_copy(e3_hbm.at[1].at[pl.ds(r0, ni)], didx)
        cps = [None] * ni
        for j in range(P):
            cps[j] = pltpu.async_copy(h_hbm.at[sidx.at[j]], rows.at[j % P],
                                      sems.at[j % P])
        for j in range(ni):
            cps[j].wait()
            pltpu.sync_copy(rows.at[j % P], acc.at[didx.at[j]], add=True)
            if j + P < ni:
                cps[j + P] = pltpu.async_copy(h_hbm.at[sidx.at[j + P]],
                                              rows.at[j % P], sems.at[j % P])
        return carry

    lax.fori_loop(0, n_outer, outer, 0)


def _segsum8_body(h_hbm, e3_hbm, zeros_hbm,
                  out0_hbm, out1_hbm, acc, sidx, didx, rows, sems):
    c = lax.axis_index("c")
    s = lax.axis_index("s")
    _zero_acc(zeros_hbm, acc, s)
    plsc.subcore_barrier()
    t_edges = E // (NC * NS)
    n_outer = t_edges // (NI8 * K)
    base_blk = (c * NS + s) * n_outer
    _edge_loop(h_hbm, e3_hbm, acc, sidx, didx, rows, sems,
               base_blk, n_outer, NI8)
    plsc.subcore_barrier()

    @pl.when(c == 0)
    def _():
        _writeback(acc, out0_hbm, s)

    @pl.when(c == 1)
    def _():
        _writeback(acc, out1_hbm, s)


def _segsum32_body(m0_hbm, m1_hbm, e3_hbm, zeros_hbm,
                   out0_hbm, out1_hbm, acc, sidx, didx, rows, sems):
    c = lax.axis_index("c")
    s = lax.axis_index("s")
    _zero_acc(zeros_hbm, acc, s)
    plsc.subcore_barrier()
    t_edges = E // NS
    n_outer = t_edges // (NI * K)
    base_blk = s * n_outer

    @pl.when(c == 0)
    def _():
        _edge_loop(m0_hbm, e3_hbm, acc, sidx, didx, rows, sems,
                   base_blk, n_outer, NI)

    @pl.when(c == 1)
    def _():
        _edge_loop(m1_hbm, e3_hbm, acc, sidx, didx, rows, sems,
                   base_blk, n_outer, NI)

    plsc.subcore_barrier()

    @pl.when(c == 0)
    def _():
        _writeback(acc, out0_hbm, s)

    @pl.when(c == 1)
    def _():
        _writeback(acc, out1_hbm, s)


def _sc_mesh():
    return plsc.VectorSubcoreMesh(core_axis_name="c", subcore_axis_name="s",
                                  num_cores=NC, num_subcores=NS)


def _segsum8(h0, e3, zeros8):
    f = pl.kernel(
        _segsum8_body,
        out_type=(jax.ShapeDtypeStruct((N, 8), jnp.float32),
                  jax.ShapeDtypeStruct((N, 8), jnp.float32)),
        mesh=_sc_mesh(),
        compiler_params=pltpu.CompilerParams(use_tc_tiling_on_sc=False),
        scratch_types=[
            pltpu.VMEM_SHARED((N, 8), jnp.float32),
            pltpu.VMEM((NI8, K), jnp.int32),
            pltpu.VMEM((NI8, K), jnp.int32),
            pltpu.VMEM((P, K, 8), jnp.float32),
            pltpu.SemaphoreType.DMA((P,)),
        ],
    )
    return f(h0, e3, zeros8)


def _segsum32(m0, m1, e3, zeros32):
    f = pl.kernel(
        _segsum32_body,
        out_type=(jax.ShapeDtypeStruct((N, 32), jnp.float32),
                  jax.ShapeDtypeStruct((N, 32), jnp.float32)),
        mesh=_sc_mesh(),
        compiler_params=pltpu.CompilerParams(use_tc_tiling_on_sc=False),
        scratch_types=[
            pltpu.VMEM_SHARED((N, 32), jnp.float32),
            pltpu.VMEM((NI, K), jnp.int32),
            pltpu.VMEM((NI, K), jnp.int32),
            pltpu.VMEM((P, K, 32), jnp.float32),
            pltpu.SemaphoreType.DMA((P,)),
        ],
    )
    return f(m0, m1, e3, zeros32)


def _emb_body(x_ref, w_ref, o_ref):
    o_ref[...] = jnp.dot(x_ref[...], w_ref[...],
                         preferred_element_type=jnp.float32)


def _tc_emb(x, w_emb):
    return pl.pallas_call(
        _emb_body,
        grid=(GRID,),
        in_specs=[pl.BlockSpec((R, 27), lambda i: (i, 0)),
                  pl.BlockSpec((27, 8), lambda i: (0, 0))],
        out_specs=pl.BlockSpec((R, 8), lambda i: (i, 0)),
        out_shape=jax.ShapeDtypeStruct((N, 8), jnp.float32),
    )(x, w_emb)


def _layer0_body(a0_ref, a1_ref, w_ref, o0_ref, o1_ref):
    h = jnp.maximum(jnp.dot(a0_ref[...] + a1_ref[...], w_ref[...],
                            preferred_element_type=jnp.float32), 0.0)
    o0_ref[...] = h[:, :32]
    o1_ref[...] = h[:, 32:]


def _tc_layer0(a0, a1, w):
    return pl.pallas_call(
        _layer0_body,
        grid=(GRID,),
        in_specs=[pl.BlockSpec((R, 8), lambda i: (i, 0)),
                  pl.BlockSpec((R, 8), lambda i: (i, 0)),
                  pl.BlockSpec((8, 64), lambda i: (0, 0))],
        out_specs=(pl.BlockSpec((R, 32), lambda i: (i, 0)),
                   pl.BlockSpec((R, 32), lambda i: (i, 0))),
        out_shape=(jax.ShapeDtypeStruct((N, 32), jnp.float32),
                   jax.ShapeDtypeStruct((N, 32), jnp.float32)),
    )(a0, a1, w)


def _layer_body(g0_ref, g1_ref, h0_ref, h1_ref, w_ref, o0_ref, o1_ref):
    g = jnp.concatenate([g0_ref[...], g1_ref[...]], axis=1)
    hp = jnp.concatenate([h0_ref[...], h1_ref[...]], axis=1)
    h = jnp.maximum(jnp.dot(g, w_ref[...],
                            preferred_element_type=jnp.float32) + hp, 0.0)
    o0_ref[...] = h[:, :32]
    o1_ref[...] = h[:, 32:]


def _tc_layer(g0, g1, h0, h1, w):
    return pl.pallas_call(
        _layer_body,
        grid=(GRID,),
        in_specs=[pl.BlockSpec((R, 32), lambda i: (i, 0))] * 4
        + [pl.BlockSpec((64, 64), lambda i: (0, 0))],
        out_specs=(pl.BlockSpec((R, 32), lambda i: (i, 0)),
                   pl.BlockSpec((R, 32), lambda i: (i, 0))),
        out_shape=(jax.ShapeDtypeStruct((N, 32), jnp.float32),
                   jax.ShapeDtypeStruct((N, 32), jnp.float32)),
    )(g0, g1, h0, h1, w)


def _final_body(g0_ref, g1_ref, h0_ref, h1_ref, w_ref, ids_ref,
                wp1_ref, wp2_ref, b_ref, o_ref, pooled):
    i = pl.program_id(0)
    g = jnp.concatenate([g0_ref[...], g1_ref[...]], axis=1)
    hp = jnp.concatenate([h0_ref[...], h1_ref[...]], axis=1)
    h = jnp.maximum(jnp.dot(g, w_ref[...],
                            preferred_element_type=jnp.float32) + hp, 0.0)
    ids = ids_ref[0]
    onehot = (lax.broadcasted_iota(jnp.int32, (B, R), 0) == ids
              ).astype(jnp.float32)

    @pl.when(i == 0)
    def _():
        pooled[...] = jnp.zeros_like(pooled)

    pooled[...] += jnp.dot(onehot, h, preferred_element_type=jnp.float32)

    @pl.when(i == GRID - 1)
    def _():
        z = jnp.maximum(jnp.dot(pooled[...], wp1_ref[...],
                                preferred_element_type=jnp.float32), 0.0)
        o_ref[...] = (jnp.dot(z, wp2_ref[...],
                              preferred_element_type=jnp.float32)
                      + b_ref[...])


def _tc_final(g0, g1, h0, h1, w, ids3, wp1, wp2, b2):
    return pl.pallas_call(
        _final_body,
        grid=(GRID,),
        in_specs=[pl.BlockSpec((R, 32), lambda i: (i, 0))] * 4
        + [pl.BlockSpec((64, 64), lambda i: (0, 0)),
           pl.BlockSpec((1, 1, R), lambda i: (i, 0, 0)),
           pl.BlockSpec((64, 32), lambda i: (0, 0)),
           pl.BlockSpec((32, 1), lambda i: (0, 0)),
           pl.BlockSpec((1, 1), lambda i: (0, 0))],
        out_specs=pl.BlockSpec((B, 1), lambda i: (0, 0)),
        out_shape=jax.ShapeDtypeStruct((B, 1), jnp.float32),
        scratch_shapes=[pltpu.VMEM((B, 64), jnp.float32)],
    )(g0, g1, h0, h1, w, ids3, wp1, wp2, b2)


def kernel(x, edge_index, graph_ids, W_emb, W_gcn0, W_gcn1, W_gcn2,
           W_p1, W_p2, b_p2):
    e3 = edge_index.reshape(2, E // K, K)
    zeros8 = jnp.zeros((WB, 8), jnp.float32)
    zeros32 = jnp.zeros((WB, 32), jnp.float32)
    ids3 = graph_ids.reshape(GRID, 1, R)
    b2 = b_p2.reshape(1, 1)

    h0 = _tc_emb(x, W_emb)
    a0, a1 = _segsum8(h0, e3, zeros8)
    h1_0, h1_1 = _tc_layer0(a0, a1, W_gcn0)
    g1_0, g1_1 = _segsum32(h1_0, h1_1, e3, zeros32)
    h2_0, h2_1 = _tc_layer(g1_0, g1_1, h1_0, h1_1, W_gcn1)
    g2_0, g2_1 = _segsum32(h2_0, h2_1, e3, zeros32)
    return _tc_final(g2_0, g2_1, h2_0, h2_1, W_gcn2, ids3, W_p1, W_p2, b2)
